# baseline (speedup 1.0000x reference)
import jax
import jax.numpy as jnp
from jax import lax
from jax.experimental import pallas as pl

E = 8
N = 8192
D_IN = 128
H = 128
EH = E * H
D_MODEL = 768
TB = 256


def _body(x_ref, m_ref, w1f_ref, b1f_ref, w2f_ref, b2e_ref, o_ref):
    xb = x_ref[...].astype(jnp.bfloat16)
    m = m_ref[...]
    z = jnp.dot(xb, w1f_ref[...], preferred_element_type=jnp.float32)
    z = jnp.maximum(z + b1f_ref[...], 0.0)
    fexp = lax.broadcasted_iota(jnp.int32, (1, EH), 1) // H
    h = jnp.where(m == fexp, z, 0.0).astype(jnp.bfloat16)
    oh = (m == lax.broadcasted_iota(jnp.int32, (TB, 16), 1)).astype(jnp.float32)
    out = jnp.dot(h, w2f_ref[...], preferred_element_type=jnp.float32)
    out = out + jnp.dot(oh, b2e_ref[...], preferred_element_type=jnp.float32)
    o_ref[...] = out


def kernel(x, W1, b1, W2, b2, tok_emb, mask):
    mask2d = mask.reshape(N, 1)
    w1f = W1.transpose(1, 0, 2).reshape(D_IN, EH).astype(jnp.bfloat16)
    b1f = b1.reshape(1, EH)
    w2f = W2.reshape(EH, D_MODEL).astype(jnp.bfloat16)
    b2e = jnp.concatenate(
        [b2, tok_emb, jnp.zeros((6, D_MODEL), jnp.float32)], axis=0)
    grid = (N // TB,)
    out = pl.pallas_call(
        _body,
        grid=grid,
        in_specs=[
            pl.BlockSpec((TB, D_IN), lambda i: (i, 0)),
            pl.BlockSpec((TB, 1), lambda i: (i, 0)),
            pl.BlockSpec((D_IN, EH), lambda i: (0, 0)),
            pl.BlockSpec((1, EH), lambda i: (0, 0)),
            pl.BlockSpec((EH, D_MODEL), lambda i: (0, 0)),
            pl.BlockSpec((16, D_MODEL), lambda i: (0, 0)),
        ],
        out_specs=pl.BlockSpec((TB, D_MODEL), lambda i: (i, 0)),
        out_shape=jax.ShapeDtypeStruct((N, D_MODEL), jnp.float32),
    )(x, mask2d, w1f, b1f, w2f, b2e)
    return out


# speedup vs baseline: 1.2738x; 1.2738x over previous
import jax
import jax.numpy as jnp
from jax import lax
from jax.experimental import pallas as pl

E = 8
N = 8192
D_IN = 128
H = 128
EH = E * H
D_MODEL = 768
TB = 1024


def _body(x_ref, m_ref, w1f_ref, b1f_ref, w2f_ref, b2e_ref, o_ref):
    xb = x_ref[...].astype(jnp.bfloat16)
    m = m_ref[...]
    z = jnp.dot(xb, w1f_ref[...], preferred_element_type=jnp.float32)
    z = jnp.maximum(z + b1f_ref[...], 0.0)
    fexp = lax.broadcasted_iota(jnp.int32, (1, EH), 1) // H
    h = jnp.where(m == fexp, z, 0.0).astype(jnp.bfloat16)
    oh = (m == lax.broadcasted_iota(jnp.int32, (TB, 16), 1)).astype(jnp.float32)
    out = jnp.dot(h, w2f_ref[...], preferred_element_type=jnp.float32)
    out = out + jnp.dot(oh, b2e_ref[...], preferred_element_type=jnp.float32)
    o_ref[...] = out


def kernel(x, W1, b1, W2, b2, tok_emb, mask):
    mask2d = mask.reshape(N, 1)
    w1f = W1.transpose(1, 0, 2).reshape(D_IN, EH).astype(jnp.bfloat16)
    b1f = b1.reshape(1, EH)
    w2f = W2.reshape(EH, D_MODEL).astype(jnp.bfloat16)
    b2e = jnp.concatenate(
        [b2, tok_emb, jnp.zeros((6, D_MODEL), jnp.float32)], axis=0)
    grid = (N // TB,)
    out = pl.pallas_call(
        _body,
        grid=grid,
        in_specs=[
            pl.BlockSpec((TB, D_IN), lambda i: (i, 0)),
            pl.BlockSpec((TB, 1), lambda i: (i, 0)),
            pl.BlockSpec((D_IN, EH), lambda i: (0, 0)),
            pl.BlockSpec((1, EH), lambda i: (0, 0)),
            pl.BlockSpec((EH, D_MODEL), lambda i: (0, 0)),
            pl.BlockSpec((16, D_MODEL), lambda i: (0, 0)),
        ],
        out_specs=pl.BlockSpec((TB, D_MODEL), lambda i: (i, 0)),
        out_shape=jax.ShapeDtypeStruct((N, D_MODEL), jnp.float32),
    )(x, mask2d, w1f, b1f, w2f, b2e)
    return out
